# baseline (speedup 1.0000x reference)
import functools

import jax
import jax.numpy as jnp
from jax.experimental import pallas as pl
from jax.experimental.pallas import tpu as pltpu

D = 768
H = 12
DH = 64
E = 8
S = 2048
SBLK = 256
EPAD = 128
NEG = -1e30
QSCALE = 0.125 * 1.4426950408889634


def _ln_qkv_body(x_ref, wq_ref, bq_ref, wk_ref, bk_ref, wv_ref, bv_ref,
                 rw_ref, rb_ref, g_ref, b_ref,
                 q_ref, k_ref, v_ref, lg_ref):
    xv = x_ref[...]
    mu = jnp.mean(xv, axis=1, keepdims=True)
    xc = xv - mu
    var = jnp.mean(xc * xc, axis=1, keepdims=True)
    xn = xc * jax.lax.rsqrt(var + 1e-5) * g_ref[...] + b_ref[...]
    xb = xn.astype(jnp.bfloat16)
    q = jnp.dot(xb, wq_ref[...], preferred_element_type=jnp.float32) + bq_ref[...]
    q_ref[...] = (q * QSCALE).astype(jnp.bfloat16)
    k_ref[...] = (jnp.dot(xb, wk_ref[...], preferred_element_type=jnp.float32)
                  + bk_ref[...]).astype(jnp.bfloat16)
    v_ref[...] = (jnp.dot(xb, wv_ref[...], preferred_element_type=jnp.float32)
                  + bv_ref[...]).astype(jnp.bfloat16)
    lg_ref[...] = jnp.dot(xn, rw_ref[...], preferred_element_type=jnp.float32) + rb_ref[...]


def _attn_body(q_ref, k_ref, v_ref, ctx_ref):
    qq = q_ref[...]
    kk = k_ref[...]
    vv = v_ref[...]
    ones = jnp.ones((S, DH), jnp.bfloat16)
    outs = []
    for h in range(2):
        q = qq[:, h * DH:(h + 1) * DH]
        k = kk[:, h * DH:(h + 1) * DH]
        v = vv[:, h * DH:(h + 1) * DH]
        s = jax.lax.dot_general(q, k, (((1,), (1,)), ((), ())),
                                preferred_element_type=jnp.float32)
        p = jnp.exp2(s.astype(jnp.bfloat16))
        c = jnp.dot(p, jnp.concatenate([v, ones], axis=1),
                    preferred_element_type=jnp.float32)
        outs.append((c[:, :DH] / c[:, DH:DH + 1]).astype(jnp.bfloat16))
    ctx_ref[...] = jnp.concatenate(outs, axis=1)


def _moe_body(ctx_ref, lg_ref, x_ref, we_ref, be_ref, out_ref):
    lg = lg_ref[...]
    col = jax.lax.broadcasted_iota(jnp.int32, lg.shape, 1)
    lg = jnp.where(col < E, lg, NEG)
    m1 = jnp.max(lg, axis=1, keepdims=True)
    i1 = jnp.min(jnp.where(lg == m1, col, EPAD), axis=1, keepdims=True)
    lg2 = jnp.where(col == i1, NEG, lg)
    m2 = jnp.max(lg2, axis=1, keepdims=True)
    i2 = jnp.min(jnp.where(lg2 == m2, col, EPAD), axis=1, keepdims=True)
    eb = jnp.exp(m2 - m1)
    den = 1.0 + eb
    g1 = 1.0 / den
    g2 = eb / den
    gates = jnp.where(col == i1, g1, 0.0) + jnp.where(col == i2, g2, 0.0)
    ctx = ctx_ref[...]
    acc = x_ref[...]
    for e in range(E):
        ge = gates[:, e:e + 1]
        acc = acc + ge * (jnp.dot(ctx, we_ref[e], preferred_element_type=jnp.float32)
                          + be_ref[e:e + 1, :])
    out_ref[...] = acc


def kernel(x, W_q, b_q, W_k, b_k, W_v, b_v, router_w, router_b,
           expert_w, expert_b, ln_gamma, ln_beta):
    xf = x.reshape(S, D)
    rw = jnp.pad(router_w, ((0, 0), (0, EPAD - E)))
    rb = jnp.pad(router_b, (0, EPAD - E)).reshape(1, EPAD)
    wq = W_q.astype(jnp.bfloat16)
    wk = W_k.astype(jnp.bfloat16)
    wv = W_v.astype(jnp.bfloat16)
    we = expert_w.astype(jnp.bfloat16)

    full = lambda *shape: pl.BlockSpec(shape, lambda i: (0,) * len(shape))
    row_blk = pl.BlockSpec((SBLK, D), lambda i: (i, 0))

    q, k, v, logits = pl.pallas_call(
        _ln_qkv_body,
        grid=(S // SBLK,),
        in_specs=[row_blk, full(D, D), full(1, D), full(D, D), full(1, D),
                  full(D, D), full(1, D), full(D, EPAD), full(1, EPAD),
                  full(1, D), full(1, D)],
        out_specs=[row_blk, row_blk, row_blk,
                   pl.BlockSpec((SBLK, EPAD), lambda i: (i, 0))],
        out_shape=[jax.ShapeDtypeStruct((S, D), jnp.bfloat16)] * 3
        + [jax.ShapeDtypeStruct((S, EPAD), jnp.float32)],
    )(xf, wq, b_q.reshape(1, D), wk, b_k.reshape(1, D),
      wv, b_v.reshape(1, D), rw, rb,
      ln_gamma.reshape(1, D), ln_beta.reshape(1, D))

    head_blk = pl.BlockSpec((S, 2 * DH), lambda g: (0, g))
    ctx = pl.pallas_call(
        _attn_body,
        grid=(H // 2,),
        in_specs=[head_blk, head_blk, head_blk],
        out_specs=head_blk,
        out_shape=jax.ShapeDtypeStruct((S, D), jnp.bfloat16),
    )(q, k, v)

    out = pl.pallas_call(
        _moe_body,
        grid=(S // SBLK,),
        in_specs=[row_blk, pl.BlockSpec((SBLK, EPAD), lambda i: (i, 0)),
                  row_blk, full(E, D, D), full(E, D)],
        out_specs=row_blk,
        out_shape=jax.ShapeDtypeStruct((S, D), jnp.float32),
    )(ctx, logits, xf, we, expert_b)

    return out.reshape(1, S, D)


# speedup vs baseline: 1.0975x; 1.0975x over previous
import functools

import jax
import jax.numpy as jnp
from jax.experimental import pallas as pl
from jax.experimental.pallas import tpu as pltpu

D = 768
H = 12
DH = 64
E = 8
S = 2048
SBLK = 256
EPAD = 128
NEG = -1e30
QSCALE = 0.125 * 1.4426950408889634


def _ln_qkv_body(x_ref, wq_ref, bq_ref, wk_ref, bk_ref, wv_ref, bv_ref,
                 rw_ref, rb_ref, g_ref, b_ref,
                 q_ref, k_ref, v_ref, lg_ref):
    xv = x_ref[...]
    mu = jnp.mean(xv, axis=1, keepdims=True)
    xc = xv - mu
    var = jnp.mean(xc * xc, axis=1, keepdims=True)
    xn = xc * jax.lax.rsqrt(var + 1e-5) * g_ref[...] + b_ref[...]
    xb = xn.astype(jnp.bfloat16)
    q = jnp.dot(xb, wq_ref[...].astype(jnp.bfloat16),
                preferred_element_type=jnp.float32) + bq_ref[...]
    q_ref[...] = (q * QSCALE).astype(jnp.bfloat16)
    k_ref[...] = (jnp.dot(xb, wk_ref[...].astype(jnp.bfloat16),
                          preferred_element_type=jnp.float32)
                  + bk_ref[...]).astype(jnp.bfloat16)
    v_ref[...] = (jnp.dot(xb, wv_ref[...].astype(jnp.bfloat16),
                          preferred_element_type=jnp.float32)
                  + bv_ref[...]).astype(jnp.bfloat16)
    lg_ref[...] = jnp.dot(xn, rw_ref[...], preferred_element_type=jnp.float32) + rb_ref[...]


def _attn_body(q_ref, k_ref, v_ref, ctx_ref):
    qq = q_ref[...]
    kk = k_ref[...]
    vv = v_ref[...]
    ones = jnp.ones((S, DH), jnp.bfloat16)
    outs = []
    for h in range(2):
        q = qq[:, h * DH:(h + 1) * DH]
        k = kk[:, h * DH:(h + 1) * DH]
        v = vv[:, h * DH:(h + 1) * DH]
        s = jax.lax.dot_general(q, k, (((1,), (1,)), ((), ())),
                                preferred_element_type=jnp.float32)
        p = jnp.exp2(s.astype(jnp.bfloat16))
        c = jnp.dot(p, jnp.concatenate([v, ones], axis=1),
                    preferred_element_type=jnp.float32)
        outs.append((c[:, :DH] / c[:, DH:DH + 1]).astype(jnp.bfloat16))
    ctx_ref[...] = jnp.concatenate(outs, axis=1)


def _moe_body(ctx_ref, lg_ref, x_ref, we_ref, be_ref, out_ref):
    lg = lg_ref[...]
    col = jax.lax.broadcasted_iota(jnp.int32, lg.shape, 1)
    lg = jnp.where(col < E, lg, NEG)
    m1 = jnp.max(lg, axis=1, keepdims=True)
    i1 = jnp.min(jnp.where(lg == m1, col, EPAD), axis=1, keepdims=True)
    lg2 = jnp.where(col == i1, NEG, lg)
    m2 = jnp.max(lg2, axis=1, keepdims=True)
    i2 = jnp.min(jnp.where(lg2 == m2, col, EPAD), axis=1, keepdims=True)
    eb = jnp.exp(m2 - m1)
    den = 1.0 + eb
    g1 = 1.0 / den
    g2 = eb / den
    gates = jnp.where(col == i1, g1, 0.0) + jnp.where(col == i2, g2, 0.0)
    ctx = ctx_ref[...]
    acc = x_ref[...]
    for e in range(E):
        ge = gates[:, e:e + 1]
        acc = acc + ge * (jnp.dot(ctx, we_ref[e].astype(jnp.bfloat16),
                                  preferred_element_type=jnp.float32)
                          + be_ref[e:e + 1, :])
    out_ref[...] = acc


def kernel(x, W_q, b_q, W_k, b_k, W_v, b_v, router_w, router_b,
           expert_w, expert_b, ln_gamma, ln_beta):
    xf = x.reshape(S, D)
    rw = jnp.pad(router_w, ((0, 0), (0, EPAD - E)))
    rb = jnp.pad(router_b, (0, EPAD - E)).reshape(1, EPAD)
    full = lambda *shape: pl.BlockSpec(shape, lambda i: (0,) * len(shape))
    row_blk = pl.BlockSpec((SBLK, D), lambda i: (i, 0))

    q, k, v, logits = pl.pallas_call(
        _ln_qkv_body,
        grid=(S // SBLK,),
        in_specs=[row_blk, full(D, D), full(1, D), full(D, D), full(1, D),
                  full(D, D), full(1, D), full(D, EPAD), full(1, EPAD),
                  full(1, D), full(1, D)],
        out_specs=[row_blk, row_blk, row_blk,
                   pl.BlockSpec((SBLK, EPAD), lambda i: (i, 0))],
        out_shape=[jax.ShapeDtypeStruct((S, D), jnp.bfloat16)] * 3
        + [jax.ShapeDtypeStruct((S, EPAD), jnp.float32)],
    )(xf, W_q, b_q.reshape(1, D), W_k, b_k.reshape(1, D),
      W_v, b_v.reshape(1, D), rw, rb,
      ln_gamma.reshape(1, D), ln_beta.reshape(1, D))

    head_blk = pl.BlockSpec((S, 2 * DH), lambda g: (0, g))
    ctx = pl.pallas_call(
        _attn_body,
        grid=(H // 2,),
        in_specs=[head_blk, head_blk, head_blk],
        out_specs=head_blk,
        out_shape=jax.ShapeDtypeStruct((S, D), jnp.bfloat16),
    )(q, k, v)

    out = pl.pallas_call(
        _moe_body,
        grid=(S // SBLK,),
        in_specs=[row_blk, pl.BlockSpec((SBLK, EPAD), lambda i: (i, 0)),
                  row_blk, full(E, D, D), full(E, D)],
        out_specs=row_blk,
        out_shape=jax.ShapeDtypeStruct((S, D), jnp.float32),
    )(ctx, logits, xf, expert_w, expert_b)

    return out.reshape(1, S, D)
